# baseline (speedup 1.0000x reference)
import functools

import jax
import jax.numpy as jnp
from jax import lax
from jax.experimental import pallas as pl
from jax.experimental.pallas import tpu as pltpu
from jax.experimental.pallas import tpu_sc as plsc

N = 10000
E = 320000
D_IN = 128
H = 8
D = 16
HD = H * D
OUT = 64

NS = 16
EPT = E // NS
B = 80
NB = EPT // B
RPS = 624
RPS_LAST = N - 15 * RPS
ZR = 16

_HIGH = lax.Precision.HIGHEST


def _dense_pre(x, fc_w, al_mat, ar_mat):
    def body(x_ref, w_ref, al_ref, ar_ref, fs_ref, eld_ref, erd_ref):
        xb = x_ref[...]
        fsb = lax.dot_general(xb, w_ref[...], (((1,), (1,)), ((), ())),
                              precision=_HIGH)
        el = jnp.dot(fsb, al_ref[...], precision=_HIGH)
        er = jnp.dot(fsb, ar_ref[...], precision=_HIGH)
        fs_ref[...] = fsb
        eld_ref[...] = jnp.concatenate([el, el], axis=1)
        erd_ref[...] = jnp.concatenate([er, er], axis=1)

    return pl.pallas_call(
        body,
        out_shape=[jax.ShapeDtypeStruct((N, HD), jnp.float32),
                   jax.ShapeDtypeStruct((N, 16), jnp.float32),
                   jax.ShapeDtypeStruct((N, 16), jnp.float32)],
    )(x, fc_w, al_mat, ar_mat)


def _lane_gather(vec, idx):
    return lax.gather(
        vec, idx[:, None],
        lax.GatherDimensionNumbers(offset_dims=(), collapsed_slice_dims=(0,),
                                   start_index_map=(0,)),
        slice_sizes=(1,), mode=lax.GatherScatterMode.PROMISE_IN_BOUNDS)


def _sc_aggregate(fs, eld, erd, pk1, pk2):
    mesh = plsc.VectorSubcoreMesh(core_axis_name="c", subcore_axis_name="s")

    @functools.partial(
        pl.kernel,
        out_type=jax.ShapeDtypeStruct((2 * N, HD), jnp.float32),
        mesh=mesh,
        compiler_params=pltpu.CompilerParams(needs_layout_passes=False,
                                             use_tc_tiling_on_sc=False),
        scratch_types=[
            pltpu.VMEM((3, B), jnp.int32),
            pltpu.VMEM((B, HD), jnp.float32),
            pltpu.VMEM((B, 16), jnp.float32),
            pltpu.VMEM((B, 16), jnp.float32),
            pltpu.SemaphoreType.DMA,
            pltpu.SemaphoreType.DMA,
            pltpu.VMEM((3, B), jnp.int32),
            pltpu.VMEM((B, HD), jnp.float32),
            pltpu.VMEM((B, 16), jnp.float32),
            pltpu.VMEM((B, 16), jnp.float32),
            pltpu.SemaphoreType.DMA,
            pltpu.SemaphoreType.DMA,
            pltpu.VMEM((ZR, HD), jnp.float32),
            pltpu.SemaphoreType.DMA,
            pltpu.VMEM_SHARED((N, HD), jnp.float32),
        ],
    )
    def k(fs_h, eld_h, erd_h, pk1_h, pk2_h, out_h,
          idx0, rows0, els0, erd0, isem0, gsem0,
          idx1, rows1, els1, erd1, isem1, gsem1,
          zbuf, zsem, accum):
        c = lax.axis_index("c")
        s = lax.axis_index("s")
        zero16 = jnp.zeros((16,), jnp.float32)
        slots = [(idx0, rows0, els0, erd0, isem0, gsem0),
                 (idx1, rows1, els1, erd1, isem1, gsem1)]

        @pl.loop(0, ZR)
        def _(r):
            for cc in range(HD // 16):
                zbuf[r, pl.ds(cc * 16, 16)] = zero16

        region = pl.multiple_of(s * RPS, 8)
        nchunks = jnp.where(s == NS - 1, RPS_LAST // ZR, RPS // ZR)

        @pl.loop(0, nchunks)
        def _(j):
            off = pl.multiple_of(region + j * ZR, 8)
            pltpu.async_copy(zbuf, accum.at[pl.ds(off, ZR)], zsem)

        @pl.loop(0, nchunks)
        def _(j):
            off = pl.multiple_of(region + j * ZR, 8)
            pltpu.make_async_copy(zbuf, accum.at[pl.ds(off, ZR)], zsem).wait()

        plsc.subcore_barrier()

        def run_graph(pk_h, out_base):
            lane2 = jnp.full((16,), 2, jnp.int32)

            def fire_idx(slot, b):
                idx_v, rows_v, els_v, erd_v, isem, gsem = slot
                base = s * EPT + b * B
                pltpu.async_copy(pk_h.at[:, pl.ds(base, B)], idx_v, isem)

            def fire_gathers(slot, b):
                idx_v, rows_v, els_v, erd_v, isem, gsem = slot
                base = s * EPT + b * B
                pltpu.make_async_copy(pk_h.at[:, pl.ds(base, B)], idx_v,
                                      isem).wait()
                pltpu.async_copy(fs_h.at[idx_v.at[0]], rows_v, gsem)
                pltpu.async_copy(eld_h.at[idx_v.at[0]], els_v, gsem)
                pltpu.async_copy(erd_h.at[idx_v.at[1]], erd_v, gsem)

            def process(slot):
                idx_v, rows_v, els_v, erd_v, isem, gsem = slot
                pltpu.make_async_copy(fs_h.at[idx_v.at[0]], rows_v, gsem).wait()
                pltpu.make_async_copy(eld_h.at[idx_v.at[0]], els_v, gsem).wait()
                pltpu.make_async_copy(erd_h.at[idx_v.at[1]], erd_v, gsem).wait()

                @plsc.parallel_loop(0, B, unroll=8)
                def _(i):
                    e = els_v[i] + erd_v[i]
                    e = jnp.maximum(e, 0.2 * e)
                    a = 1.0 / (1.0 + jnp.exp(-e))
                    w = plsc.bitcast(
                        plsc.load_gather(
                            idx_v, [lane2, jnp.full((16,), i, jnp.int32)]),
                        jnp.float32)
                    a = a * w
                    for h in range(H):
                        ah = _lane_gather(a, jnp.full((16,), h, jnp.int32))
                        rows_v[i, pl.ds(h * D, D)] = rows_v[i, pl.ds(h * D, D)] * ah

                pltpu.sync_copy(rows_v, accum.at[idx_v.at[1]], add=True)

            fire_idx(slots[0], 0)
            fire_gathers(slots[0], 0)
            fire_idx(slots[1], 1)

            @pl.loop(0, NB, step=2)
            def _(b):
                fire_gathers(slots[1], b + 1)
                process(slots[0])

                @pl.when(b + 2 < NB)
                def _():
                    fire_idx(slots[0], b + 2)
                    fire_gathers(slots[0], b + 2)

                process(slots[1])

                @pl.when(b + 3 < NB)
                def _():
                    fire_idx(slots[1], b + 3)

            plsc.subcore_barrier()
            reg = pl.multiple_of(s * RPS, 8)

            @pl.when(s < NS - 1)
            def _():
                pltpu.sync_copy(accum.at[pl.ds(reg, RPS)],
                                out_h.at[pl.ds(out_base + reg, RPS)])

            @pl.when(s == NS - 1)
            def _():
                lastoff = (NS - 1) * RPS
                pltpu.sync_copy(accum.at[pl.ds(lastoff, RPS_LAST)],
                                out_h.at[pl.ds(out_base + lastoff, RPS_LAST)])

        @pl.when(c == 0)
        def _():
            run_graph(pk1_h, 0)

        @pl.when(c == 1)
        def _():
            run_graph(pk2_h, N)

    return k(fs, eld, erd, pk1, pk2)


def _dense_post(acc, bias_g, sa_w1, sa_b1, sa_w2, pred_w, pred_b):
    def body(acc_ref, bg_ref, w1_ref, b1_ref, w2_ref, pw_ref, pb_ref, out_ref):
        bg = bg_ref[...]
        z1 = acc_ref[:N, :] + bg[None, :]
        z2 = acc_ref[N:, :] + bg[None, :]
        z1 = jnp.where(z1 > 0, z1, jnp.exp(z1) - 1.0)
        z2 = jnp.where(z2 > 0, z2, jnp.exp(z2) - 1.0)
        t1 = jnp.tanh(lax.dot_general(z1, w1_ref[...], (((1,), (1,)), ((), ())),
                                      precision=_HIGH) + b1_ref[...][None, :])
        t2 = jnp.tanh(lax.dot_general(z2, w1_ref[...], (((1,), (1,)), ((), ())),
                                      precision=_HIGH) + b1_ref[...][None, :])
        w2row = w2_ref[...][0]
        s1 = jnp.sum(t1 * w2row[None, :]) / N
        s2 = jnp.sum(t2 * w2row[None, :]) / N
        m = jnp.maximum(s1, s2)
        e1 = jnp.exp(s1 - m)
        e2 = jnp.exp(s2 - m)
        b1 = e1 / (e1 + e2)
        b2 = e2 / (e1 + e2)
        hfin = b1 * z1 + b2 * z2
        out_ref[...] = lax.dot_general(hfin, pw_ref[...], (((1,), (1,)), ((), ())),
                                       precision=_HIGH) + pb_ref[...][None, :]

    return pl.pallas_call(
        body,
        out_shape=jax.ShapeDtypeStruct((N, OUT), jnp.float32),
    )(acc, bias_g, sa_w1, sa_b1, sa_w2, pred_w, pred_b)


def kernel(x, edge_index1, edge_weight1, edge_index2, edge_weight2, fc_w,
           attn_l, attn_r, bias_g, sa_w1, sa_b1, sa_w2, pred_w, pred_b):
    head_of = jnp.arange(HD, dtype=jnp.int32)[:, None] // D
    mask = (head_of == jnp.arange(H, dtype=jnp.int32)[None, :]).astype(jnp.float32)
    al_mat = attn_l.reshape(HD)[:, None] * mask
    ar_mat = attn_r.reshape(HD)[:, None] * mask

    fs, eld, erd = _dense_pre(x, fc_w, al_mat, ar_mat)
    pk1 = jnp.concatenate(
        [edge_index1,
         lax.bitcast_convert_type(edge_weight1, jnp.int32)[None, :]], axis=0)
    pk2 = jnp.concatenate(
        [edge_index2,
         lax.bitcast_convert_type(edge_weight2, jnp.int32)[None, :]], axis=0)
    acc = _sc_aggregate(fs, eld, erd, pk1, pk2)
    return _dense_post(acc, bias_g, sa_w1, sa_b1, sa_w2, pred_w, pred_b)


# speedup vs baseline: 1.6179x; 1.6179x over previous
import functools

import jax
import jax.numpy as jnp
from jax import lax
from jax.experimental import pallas as pl
from jax.experimental.pallas import tpu as pltpu
from jax.experimental.pallas import tpu_sc as plsc

N = 10000
E = 320000
D_IN = 128
H = 8
D = 16
HD = H * D
OUT = 64

NS = 16
EPT = E // NS
B = 80
NB = EPT // B
RPS = 624
RPS_LAST = N - 15 * RPS
ZR = 16

_HIGH = lax.Precision.HIGHEST


def _dense_pre(x, fc_w, al_mat, ar_mat):
    def body(x_ref, w_ref, al_ref, ar_ref, fs_ref, eld_ref, erd_ref):
        xb = x_ref[...]
        fsb = lax.dot_general(xb, w_ref[...], (((1,), (1,)), ((), ())),
                              precision=_HIGH)
        el = jnp.dot(fsb, al_ref[...], precision=_HIGH)
        er = jnp.dot(fsb, ar_ref[...], precision=_HIGH)
        fs_ref[...] = fsb
        eld_ref[...] = jnp.concatenate([el, el], axis=1)
        erd_ref[...] = jnp.concatenate([er, er], axis=1)

    return pl.pallas_call(
        body,
        out_shape=[jax.ShapeDtypeStruct((N, HD), jnp.float32),
                   jax.ShapeDtypeStruct((N, 16), jnp.float32),
                   jax.ShapeDtypeStruct((N, 16), jnp.float32)],
    )(x, fc_w, al_mat, ar_mat)


def _lane_gather(vec, idx):
    return lax.gather(
        vec, idx[:, None],
        lax.GatherDimensionNumbers(offset_dims=(), collapsed_slice_dims=(0,),
                                   start_index_map=(0,)),
        slice_sizes=(1,), mode=lax.GatherScatterMode.PROMISE_IN_BOUNDS)


def _sc_aggregate(fs, eld, erd, pk1, pk2):
    mesh = plsc.VectorSubcoreMesh(core_axis_name="c", subcore_axis_name="s")

    @functools.partial(
        pl.kernel,
        out_type=jax.ShapeDtypeStruct((2 * N, HD), jnp.float32),
        mesh=mesh,
        compiler_params=pltpu.CompilerParams(needs_layout_passes=False,
                                             use_tc_tiling_on_sc=False),
        scratch_types=[
            pltpu.VMEM((3, B), jnp.int32),
            pltpu.VMEM((B, HD), jnp.float32),
            pltpu.VMEM((B, 16), jnp.float32),
            pltpu.VMEM((B, 16), jnp.float32),
            pltpu.SemaphoreType.DMA,
            pltpu.SemaphoreType.DMA,
            pltpu.VMEM((3, B), jnp.int32),
            pltpu.VMEM((B, HD), jnp.float32),
            pltpu.VMEM((B, 16), jnp.float32),
            pltpu.VMEM((B, 16), jnp.float32),
            pltpu.SemaphoreType.DMA,
            pltpu.SemaphoreType.DMA,
            pltpu.VMEM((ZR, HD), jnp.float32),
            pltpu.SemaphoreType.DMA,
            pltpu.VMEM_SHARED((N, HD), jnp.float32),
        ],
    )
    def k(fs_h, eld_h, erd_h, pk1_h, pk2_h, out_h,
          idx0, rows0, els0, erd0, isem0, gsem0,
          idx1, rows1, els1, erd1, isem1, gsem1,
          zbuf, zsem, accum):
        c = lax.axis_index("c")
        s = lax.axis_index("s")
        zero16 = jnp.zeros((16,), jnp.float32)
        slots = [(idx0, rows0, els0, erd0, isem0, gsem0),
                 (idx1, rows1, els1, erd1, isem1, gsem1)]

        @pl.loop(0, ZR)
        def _(r):
            for cc in range(HD // 16):
                zbuf[r, pl.ds(cc * 16, 16)] = zero16

        region = pl.multiple_of(s * RPS, 8)
        nchunks = jnp.where(s == NS - 1, RPS_LAST // ZR, RPS // ZR)

        @pl.loop(0, nchunks)
        def _(j):
            off = pl.multiple_of(region + j * ZR, 8)
            pltpu.async_copy(zbuf, accum.at[pl.ds(off, ZR)], zsem)

        @pl.loop(0, nchunks)
        def _(j):
            off = pl.multiple_of(region + j * ZR, 8)
            pltpu.make_async_copy(zbuf, accum.at[pl.ds(off, ZR)], zsem).wait()

        plsc.subcore_barrier()

        def run_graph(pk_h, out_base):
            lane2 = jnp.full((16,), 2, jnp.int32)

            def fire_idx(slot, b):
                idx_v, rows_v, els_v, erd_v, isem, gsem = slot
                base = s * EPT + b * B
                pltpu.async_copy(pk_h.at[:, pl.ds(base, B)], idx_v, isem)

            def fire_gathers(slot, b):
                idx_v, rows_v, els_v, erd_v, isem, gsem = slot
                base = s * EPT + b * B
                pltpu.make_async_copy(pk_h.at[:, pl.ds(base, B)], idx_v,
                                      isem).wait()
                pltpu.async_copy(fs_h.at[idx_v.at[0]], rows_v, gsem)
                pltpu.async_copy(eld_h.at[idx_v.at[0]], els_v, gsem)
                pltpu.async_copy(erd_h.at[idx_v.at[1]], erd_v, gsem)

            def process(slot):
                idx_v, rows_v, els_v, erd_v, isem, gsem = slot
                pltpu.make_async_copy(fs_h.at[idx_v.at[0]], rows_v, gsem).wait()
                pltpu.make_async_copy(eld_h.at[idx_v.at[0]], els_v, gsem).wait()
                pltpu.make_async_copy(erd_h.at[idx_v.at[1]], erd_v, gsem).wait()

                @plsc.parallel_loop(0, B, unroll=2)
                def _(i):
                    e = els_v[i] + erd_v[i]
                    e = jnp.maximum(e, 0.2 * e)
                    a = 1.0 / (1.0 + jnp.exp(-e))
                    w = plsc.bitcast(
                        plsc.load_gather(
                            idx_v, [lane2, jnp.full((16,), i, jnp.int32)]),
                        jnp.float32)
                    a = a * w
                    for h in range(H):
                        ah = _lane_gather(a, jnp.full((16,), h, jnp.int32))
                        rows_v[i, pl.ds(h * D, D)] = rows_v[i, pl.ds(h * D, D)] * ah

                pltpu.sync_copy(rows_v, accum.at[idx_v.at[1]], add=True)

            fire_idx(slots[0], 0)
            fire_gathers(slots[0], 0)
            fire_idx(slots[1], 1)

            @pl.loop(0, NB, step=2)
            def _(b):
                fire_gathers(slots[1], b + 1)
                process(slots[0])

                @pl.when(b + 2 < NB)
                def _():
                    fire_idx(slots[0], b + 2)
                    fire_gathers(slots[0], b + 2)

                process(slots[1])

                @pl.when(b + 3 < NB)
                def _():
                    fire_idx(slots[1], b + 3)

            plsc.subcore_barrier()
            reg = pl.multiple_of(s * RPS, 8)

            @pl.when(s < NS - 1)
            def _():
                pltpu.sync_copy(accum.at[pl.ds(reg, RPS)],
                                out_h.at[pl.ds(out_base + reg, RPS)])

            @pl.when(s == NS - 1)
            def _():
                lastoff = (NS - 1) * RPS
                pltpu.sync_copy(accum.at[pl.ds(lastoff, RPS_LAST)],
                                out_h.at[pl.ds(out_base + lastoff, RPS_LAST)])

        @pl.when(c == 0)
        def _():
            run_graph(pk1_h, 0)

        @pl.when(c == 1)
        def _():
            run_graph(pk2_h, N)

    return k(fs, eld, erd, pk1, pk2)


def _dense_post(acc, bias_g, sa_w1, sa_b1, sa_w2, pred_w, pred_b):
    def body(acc_ref, bg_ref, w1_ref, b1_ref, w2_ref, pw_ref, pb_ref, out_ref):
        bg = bg_ref[...]
        z1 = acc_ref[:N, :] + bg[None, :]
        z2 = acc_ref[N:, :] + bg[None, :]
        z1 = jnp.where(z1 > 0, z1, jnp.exp(z1) - 1.0)
        z2 = jnp.where(z2 > 0, z2, jnp.exp(z2) - 1.0)
        t1 = jnp.tanh(lax.dot_general(z1, w1_ref[...], (((1,), (1,)), ((), ())),
                                      precision=_HIGH) + b1_ref[...][None, :])
        t2 = jnp.tanh(lax.dot_general(z2, w1_ref[...], (((1,), (1,)), ((), ())),
                                      precision=_HIGH) + b1_ref[...][None, :])
        w2row = w2_ref[...][0]
        s1 = jnp.sum(t1 * w2row[None, :]) / N
        s2 = jnp.sum(t2 * w2row[None, :]) / N
        m = jnp.maximum(s1, s2)
        e1 = jnp.exp(s1 - m)
        e2 = jnp.exp(s2 - m)
        b1 = e1 / (e1 + e2)
        b2 = e2 / (e1 + e2)
        hfin = b1 * z1 + b2 * z2
        out_ref[...] = lax.dot_general(hfin, pw_ref[...], (((1,), (1,)), ((), ())),
                                       precision=_HIGH) + pb_ref[...][None, :]

    return pl.pallas_call(
        body,
        out_shape=jax.ShapeDtypeStruct((N, OUT), jnp.float32),
    )(acc, bias_g, sa_w1, sa_b1, sa_w2, pred_w, pred_b)


def kernel(x, edge_index1, edge_weight1, edge_index2, edge_weight2, fc_w,
           attn_l, attn_r, bias_g, sa_w1, sa_b1, sa_w2, pred_w, pred_b):
    head_of = jnp.arange(HD, dtype=jnp.int32)[:, None] // D
    mask = (head_of == jnp.arange(H, dtype=jnp.int32)[None, :]).astype(jnp.float32)
    al_mat = attn_l.reshape(HD)[:, None] * mask
    ar_mat = attn_r.reshape(HD)[:, None] * mask

    fs, eld, erd = _dense_pre(x, fc_w, al_mat, ar_mat)
    pk1 = jnp.concatenate(
        [edge_index1,
         lax.bitcast_convert_type(edge_weight1, jnp.int32)[None, :]], axis=0)
    pk2 = jnp.concatenate(
        [edge_index2,
         lax.bitcast_convert_type(edge_weight2, jnp.int32)[None, :]], axis=0)
    acc = _sc_aggregate(fs, eld, erd, pk1, pk2)
    return _dense_post(acc, bias_g, sa_w1, sa_b1, sa_w2, pred_w, pred_b)


# speedup vs baseline: 1.6193x; 1.0008x over previous
import functools

import jax
import jax.numpy as jnp
from jax import lax
from jax.experimental import pallas as pl
from jax.experimental.pallas import tpu as pltpu
from jax.experimental.pallas import tpu_sc as plsc

N = 10000
E = 320000
D_IN = 128
H = 8
D = 16
HD = H * D
OUT = 64

NS = 16
EPT = E // NS
B = 80
NB = EPT // B
RPS = 624
RPS_LAST = N - 15 * RPS
ZR = 16

_HIGH = lax.Precision.HIGHEST


def _dense_pre(x, fc_w, al_mat, ar_mat):
    def body(x_ref, w_ref, al_ref, ar_ref, fs_ref, eld_ref, erd_ref):
        xb = x_ref[...]
        fsb = lax.dot_general(xb, w_ref[...], (((1,), (1,)), ((), ())),
                              precision=_HIGH)
        el = jnp.dot(fsb, al_ref[...], precision=_HIGH)
        er = jnp.dot(fsb, ar_ref[...], precision=_HIGH)
        fs_ref[...] = fsb
        eld_ref[...] = jnp.concatenate([el, el], axis=1)
        erd_ref[...] = jnp.concatenate([er, er], axis=1)

    return pl.pallas_call(
        body,
        out_shape=[jax.ShapeDtypeStruct((N, HD), jnp.float32),
                   jax.ShapeDtypeStruct((N, 16), jnp.float32),
                   jax.ShapeDtypeStruct((N, 16), jnp.float32)],
    )(x, fc_w, al_mat, ar_mat)


def _lane_gather(vec, idx):
    return lax.gather(
        vec, idx[:, None],
        lax.GatherDimensionNumbers(offset_dims=(), collapsed_slice_dims=(0,),
                                   start_index_map=(0,)),
        slice_sizes=(1,), mode=lax.GatherScatterMode.PROMISE_IN_BOUNDS)


def _sc_aggregate(fs, eld, erd, pk1, pk2):
    mesh = plsc.VectorSubcoreMesh(core_axis_name="c", subcore_axis_name="s")

    @functools.partial(
        pl.kernel,
        out_type=jax.ShapeDtypeStruct((2 * N, HD), jnp.float32),
        mesh=mesh,
        compiler_params=pltpu.CompilerParams(needs_layout_passes=False,
                                             use_tc_tiling_on_sc=False),
        scratch_types=[
            pltpu.VMEM((3, B), jnp.int32),
            pltpu.VMEM((B, HD), jnp.float32),
            pltpu.VMEM((B, 16), jnp.float32),
            pltpu.VMEM((B, 16), jnp.float32),
            pltpu.SemaphoreType.DMA,
            pltpu.SemaphoreType.DMA,
            pltpu.VMEM((3, B), jnp.int32),
            pltpu.VMEM((B, HD), jnp.float32),
            pltpu.VMEM((B, 16), jnp.float32),
            pltpu.VMEM((B, 16), jnp.float32),
            pltpu.SemaphoreType.DMA,
            pltpu.SemaphoreType.DMA,
            pltpu.VMEM((ZR, HD), jnp.float32),
            pltpu.SemaphoreType.DMA,
            pltpu.VMEM_SHARED((N, HD), jnp.float32),
        ],
    )
    def k(fs_h, eld_h, erd_h, pk1_h, pk2_h, out_h,
          idx0, rows0, els0, erd0, isem0, gsem0,
          idx1, rows1, els1, erd1, isem1, gsem1,
          zbuf, zsem, accum):
        c = lax.axis_index("c")
        s = lax.axis_index("s")
        zero16 = jnp.zeros((16,), jnp.float32)
        slots = [(idx0, rows0, els0, erd0, isem0, gsem0),
                 (idx1, rows1, els1, erd1, isem1, gsem1)]

        @pl.loop(0, ZR)
        def _(r):
            for cc in range(HD // 16):
                zbuf[r, pl.ds(cc * 16, 16)] = zero16

        region = pl.multiple_of(s * RPS, 8)
        nchunks = jnp.where(s == NS - 1, RPS_LAST // ZR, RPS // ZR)

        @pl.loop(0, nchunks)
        def _(j):
            off = pl.multiple_of(region + j * ZR, 8)
            pltpu.async_copy(zbuf, accum.at[pl.ds(off, ZR)], zsem)

        @pl.loop(0, nchunks)
        def _(j):
            off = pl.multiple_of(region + j * ZR, 8)
            pltpu.make_async_copy(zbuf, accum.at[pl.ds(off, ZR)], zsem).wait()

        plsc.subcore_barrier()

        def run_graph(pk_h, out_base):
            lane2 = jnp.full((16,), 2, jnp.int32)

            def fire_idx(slot, b):
                idx_v, rows_v, els_v, erd_v, isem, gsem = slot
                base = s * EPT + b * B
                pltpu.async_copy(pk_h.at[:, pl.ds(base, B)], idx_v, isem)

            def fire_gathers(slot, b):
                idx_v, rows_v, els_v, erd_v, isem, gsem = slot
                base = s * EPT + b * B
                pltpu.make_async_copy(pk_h.at[:, pl.ds(base, B)], idx_v,
                                      isem).wait()
                pltpu.async_copy(fs_h.at[idx_v.at[0]], rows_v, gsem)
                pltpu.async_copy(eld_h.at[idx_v.at[0]], els_v, gsem)
                pltpu.async_copy(erd_h.at[idx_v.at[1]], erd_v, gsem)

            def process(slot):
                idx_v, rows_v, els_v, erd_v, isem, gsem = slot
                pltpu.make_async_copy(fs_h.at[idx_v.at[0]], rows_v, gsem).wait()
                pltpu.make_async_copy(eld_h.at[idx_v.at[0]], els_v, gsem).wait()
                pltpu.make_async_copy(erd_h.at[idx_v.at[1]], erd_v, gsem).wait()

                @plsc.parallel_loop(0, B, unroll=4)
                def _(i):
                    e = els_v[i] + erd_v[i]
                    e = jnp.maximum(e, 0.2 * e)
                    a = 1.0 / (1.0 + jnp.exp(-e))
                    w = plsc.bitcast(
                        plsc.load_gather(
                            idx_v, [lane2, jnp.full((16,), i, jnp.int32)]),
                        jnp.float32)
                    a = a * w
                    for h in range(H):
                        ah = _lane_gather(a, jnp.full((16,), h, jnp.int32))
                        rows_v[i, pl.ds(h * D, D)] = rows_v[i, pl.ds(h * D, D)] * ah

                pltpu.sync_copy(rows_v, accum.at[idx_v.at[1]], add=True)

            fire_idx(slots[0], 0)
            fire_gathers(slots[0], 0)
            fire_idx(slots[1], 1)

            @pl.loop(0, NB, step=2)
            def _(b):
                fire_gathers(slots[1], b + 1)
                process(slots[0])

                @pl.when(b + 2 < NB)
                def _():
                    fire_idx(slots[0], b + 2)
                    fire_gathers(slots[0], b + 2)

                process(slots[1])

                @pl.when(b + 3 < NB)
                def _():
                    fire_idx(slots[1], b + 3)

            plsc.subcore_barrier()
            reg = pl.multiple_of(s * RPS, 8)

            @pl.when(s < NS - 1)
            def _():
                pltpu.sync_copy(accum.at[pl.ds(reg, RPS)],
                                out_h.at[pl.ds(out_base + reg, RPS)])

            @pl.when(s == NS - 1)
            def _():
                lastoff = (NS - 1) * RPS
                pltpu.sync_copy(accum.at[pl.ds(lastoff, RPS_LAST)],
                                out_h.at[pl.ds(out_base + lastoff, RPS_LAST)])

        @pl.when(c == 0)
        def _():
            run_graph(pk1_h, 0)

        @pl.when(c == 1)
        def _():
            run_graph(pk2_h, N)

    return k(fs, eld, erd, pk1, pk2)


def _dense_post(acc, bias_g, sa_w1, sa_b1, sa_w2, pred_w, pred_b):
    def body(acc_ref, bg_ref, w1_ref, b1_ref, w2_ref, pw_ref, pb_ref, out_ref):
        bg = bg_ref[...]
        z1 = acc_ref[:N, :] + bg[None, :]
        z2 = acc_ref[N:, :] + bg[None, :]
        z1 = jnp.where(z1 > 0, z1, jnp.exp(z1) - 1.0)
        z2 = jnp.where(z2 > 0, z2, jnp.exp(z2) - 1.0)
        t1 = jnp.tanh(lax.dot_general(z1, w1_ref[...], (((1,), (1,)), ((), ())),
                                      precision=_HIGH) + b1_ref[...][None, :])
        t2 = jnp.tanh(lax.dot_general(z2, w1_ref[...], (((1,), (1,)), ((), ())),
                                      precision=_HIGH) + b1_ref[...][None, :])
        w2row = w2_ref[...][0]
        s1 = jnp.sum(t1 * w2row[None, :]) / N
        s2 = jnp.sum(t2 * w2row[None, :]) / N
        m = jnp.maximum(s1, s2)
        e1 = jnp.exp(s1 - m)
        e2 = jnp.exp(s2 - m)
        b1 = e1 / (e1 + e2)
        b2 = e2 / (e1 + e2)
        hfin = b1 * z1 + b2 * z2
        out_ref[...] = lax.dot_general(hfin, pw_ref[...], (((1,), (1,)), ((), ())),
                                       precision=_HIGH) + pb_ref[...][None, :]

    return pl.pallas_call(
        body,
        out_shape=jax.ShapeDtypeStruct((N, OUT), jnp.float32),
    )(acc, bias_g, sa_w1, sa_b1, sa_w2, pred_w, pred_b)


def kernel(x, edge_index1, edge_weight1, edge_index2, edge_weight2, fc_w,
           attn_l, attn_r, bias_g, sa_w1, sa_b1, sa_w2, pred_w, pred_b):
    head_of = jnp.arange(HD, dtype=jnp.int32)[:, None] // D
    mask = (head_of == jnp.arange(H, dtype=jnp.int32)[None, :]).astype(jnp.float32)
    al_mat = attn_l.reshape(HD)[:, None] * mask
    ar_mat = attn_r.reshape(HD)[:, None] * mask

    fs, eld, erd = _dense_pre(x, fc_w, al_mat, ar_mat)
    pk1 = jnp.concatenate(
        [edge_index1,
         lax.bitcast_convert_type(edge_weight1, jnp.int32)[None, :]], axis=0)
    pk2 = jnp.concatenate(
        [edge_index2,
         lax.bitcast_convert_type(edge_weight2, jnp.int32)[None, :]], axis=0)
    acc = _sc_aggregate(fs, eld, erd, pk1, pk2)
    return _dense_post(acc, bias_g, sa_w1, sa_b1, sa_w2, pred_w, pred_b)


# speedup vs baseline: 1.8838x; 1.1634x over previous
import functools

import jax
import jax.numpy as jnp
from jax import lax
from jax.experimental import pallas as pl
from jax.experimental.pallas import tpu as pltpu
from jax.experimental.pallas import tpu_sc as plsc

N = 10000
E = 320000
D_IN = 128
H = 8
D = 16
HD = H * D
OUT = 64

NS = 16
EPT = E // NS
B = 80
NB = EPT // B
RPS = 624
RPS_LAST = N - 15 * RPS
ZR = 16

_HIGH = lax.Precision.HIGHEST


def _dense_pre(x, fc_w, al_mat, ar_mat):
    def body(x_ref, w_ref, al_ref, ar_ref, fs_ref, eld_ref, erd_ref):
        xb = x_ref[...]
        fsb = lax.dot_general(xb, w_ref[...], (((1,), (1,)), ((), ())),
                              precision=_HIGH)
        el = jnp.dot(fsb, al_ref[...], precision=_HIGH)
        er = jnp.dot(fsb, ar_ref[...], precision=_HIGH)
        fs_ref[...] = fsb
        eld_ref[...] = jnp.concatenate([el, el], axis=1)
        erd_ref[...] = jnp.concatenate([er, er], axis=1)

    return pl.pallas_call(
        body,
        out_shape=[jax.ShapeDtypeStruct((N, HD), jnp.float32),
                   jax.ShapeDtypeStruct((N, 16), jnp.float32),
                   jax.ShapeDtypeStruct((N, 16), jnp.float32)],
    )(x, fc_w, al_mat, ar_mat)


def _lane_gather(vec, idx):
    return lax.gather(
        vec, idx[:, None],
        lax.GatherDimensionNumbers(offset_dims=(), collapsed_slice_dims=(0,),
                                   start_index_map=(0,)),
        slice_sizes=(1,), mode=lax.GatherScatterMode.PROMISE_IN_BOUNDS)


def _sc_aggregate(fs, eld, erd, pk1, pk2):
    mesh = plsc.VectorSubcoreMesh(core_axis_name="c", subcore_axis_name="s")

    @functools.partial(
        pl.kernel,
        out_type=jax.ShapeDtypeStruct((2 * N, HD), jnp.float32),
        mesh=mesh,
        compiler_params=pltpu.CompilerParams(needs_layout_passes=False,
                                             use_tc_tiling_on_sc=False),
        scratch_types=[
            pltpu.VMEM((3, B), jnp.int32),
            pltpu.VMEM((B, HD), jnp.float32),
            pltpu.VMEM((B, 16), jnp.float32),
            pltpu.VMEM((B, 16), jnp.float32),
            pltpu.SemaphoreType.DMA,
            pltpu.SemaphoreType.DMA,
            pltpu.SemaphoreType.DMA,
            pltpu.VMEM((3, B), jnp.int32),
            pltpu.VMEM((B, HD), jnp.float32),
            pltpu.VMEM((B, 16), jnp.float32),
            pltpu.VMEM((B, 16), jnp.float32),
            pltpu.SemaphoreType.DMA,
            pltpu.SemaphoreType.DMA,
            pltpu.SemaphoreType.DMA,
            pltpu.VMEM((3, B), jnp.int32),
            pltpu.VMEM((B, HD), jnp.float32),
            pltpu.VMEM((B, 16), jnp.float32),
            pltpu.VMEM((B, 16), jnp.float32),
            pltpu.SemaphoreType.DMA,
            pltpu.SemaphoreType.DMA,
            pltpu.SemaphoreType.DMA,
            pltpu.VMEM((ZR, HD), jnp.float32),
            pltpu.SemaphoreType.DMA,
            pltpu.VMEM_SHARED((N, HD), jnp.float32),
        ],
    )
    def k(fs_h, eld_h, erd_h, pk1_h, pk2_h, out_h,
          idx0, rows0, els0, erd0, isem0, gsem0, ssem0,
          idx1, rows1, els1, erd1, isem1, gsem1, ssem1,
          idx2, rows2, els2, erd2, isem2, gsem2, ssem2,
          zbuf, zsem, accum):
        c = lax.axis_index("c")
        s = lax.axis_index("s")
        zero16 = jnp.zeros((16,), jnp.float32)
        slots = [(idx0, rows0, els0, erd0, isem0, gsem0, ssem0),
                 (idx1, rows1, els1, erd1, isem1, gsem1, ssem1),
                 (idx2, rows2, els2, erd2, isem2, gsem2, ssem2)]

        @pl.loop(0, ZR)
        def _(r):
            for cc in range(HD // 16):
                zbuf[r, pl.ds(cc * 16, 16)] = zero16

        region = pl.multiple_of(s * RPS, 8)
        nchunks = jnp.where(s == NS - 1, RPS_LAST // ZR, RPS // ZR)

        @pl.loop(0, nchunks)
        def _(j):
            off = pl.multiple_of(region + j * ZR, 8)
            pltpu.async_copy(zbuf, accum.at[pl.ds(off, ZR)], zsem)

        @pl.loop(0, nchunks)
        def _(j):
            off = pl.multiple_of(region + j * ZR, 8)
            pltpu.make_async_copy(zbuf, accum.at[pl.ds(off, ZR)], zsem).wait()

        plsc.subcore_barrier()

        def run_graph(pk_h, out_base):
            lane2 = jnp.full((16,), 2, jnp.int32)

            def fire_idx(slot, b):
                idx_v, rows_v, els_v, erd_v, isem, gsem, ssem = slot
                base = s * EPT + b * B
                pltpu.async_copy(pk_h.at[:, pl.ds(base, B)], idx_v, isem)

            def fire_gathers(slot, b):
                idx_v, rows_v, els_v, erd_v, isem, gsem, ssem = slot
                base = s * EPT + b * B
                pltpu.make_async_copy(pk_h.at[:, pl.ds(base, B)], idx_v,
                                      isem).wait()
                pltpu.async_copy(fs_h.at[idx_v.at[0]], rows_v, gsem)
                pltpu.async_copy(eld_h.at[idx_v.at[0]], els_v, gsem)
                pltpu.async_copy(erd_h.at[idx_v.at[1]], erd_v, gsem)

            def wait_scatter(slot):
                idx_v, rows_v, els_v, erd_v, isem, gsem, ssem = slot
                pltpu.make_async_copy(rows_v, accum.at[idx_v.at[1]],
                                      ssem).wait()

            def process(slot):
                idx_v, rows_v, els_v, erd_v, isem, gsem, ssem = slot
                pltpu.make_async_copy(fs_h.at[idx_v.at[0]], rows_v, gsem).wait()
                pltpu.make_async_copy(eld_h.at[idx_v.at[0]], els_v, gsem).wait()
                pltpu.make_async_copy(erd_h.at[idx_v.at[1]], erd_v, gsem).wait()

                @plsc.parallel_loop(0, B, unroll=4)
                def _(i):
                    e = els_v[i] + erd_v[i]
                    e = jnp.maximum(e, 0.2 * e)
                    a = 1.0 / (1.0 + jnp.exp(-e))
                    w = plsc.bitcast(
                        plsc.load_gather(
                            idx_v, [lane2, jnp.full((16,), i, jnp.int32)]),
                        jnp.float32)
                    a = a * w
                    for h in range(H):
                        ah = _lane_gather(a, jnp.full((16,), h, jnp.int32))
                        rows_v[i, pl.ds(h * D, D)] = rows_v[i, pl.ds(h * D, D)] * ah

                pltpu.async_copy(rows_v, accum.at[idx_v.at[1]], ssem, add=True)

            fire_idx(slots[0], 0)
            fire_gathers(slots[0], 0)
            fire_idx(slots[1], 1)
            fire_gathers(slots[1], 1)

            @pl.loop(0, NB, step=3)
            def _(b):
                for k in range(3):
                    t = b + k
                    cur = slots[k]
                    nxt = slots[(k + 2) % 3]

                    @pl.when(t < NB)
                    def _():
                        process(cur)

                        @pl.when(t + 2 < NB)
                        def _():
                            @pl.when(t >= 1)
                            def _():
                                wait_scatter(nxt)

                            fire_idx(nxt, t + 2)
                            fire_gathers(nxt, t + 2)

            for sl in slots:
                wait_scatter(sl)

            plsc.subcore_barrier()
            reg = pl.multiple_of(s * RPS, 8)

            @pl.when(s < NS - 1)
            def _():
                pltpu.sync_copy(accum.at[pl.ds(reg, RPS)],
                                out_h.at[pl.ds(out_base + reg, RPS)])

            @pl.when(s == NS - 1)
            def _():
                lastoff = (NS - 1) * RPS
                pltpu.sync_copy(accum.at[pl.ds(lastoff, RPS_LAST)],
                                out_h.at[pl.ds(out_base + lastoff, RPS_LAST)])

        @pl.when(c == 0)
        def _():
            run_graph(pk1_h, 0)

        @pl.when(c == 1)
        def _():
            run_graph(pk2_h, N)

    return k(fs, eld, erd, pk1, pk2)


def _dense_post(acc, bias_g, sa_w1, sa_b1, sa_w2, pred_w, pred_b):
    def body(acc_ref, bg_ref, w1_ref, b1_ref, w2_ref, pw_ref, pb_ref, out_ref):
        bg = bg_ref[...]
        z1 = acc_ref[:N, :] + bg[None, :]
        z2 = acc_ref[N:, :] + bg[None, :]
        z1 = jnp.where(z1 > 0, z1, jnp.exp(z1) - 1.0)
        z2 = jnp.where(z2 > 0, z2, jnp.exp(z2) - 1.0)
        t1 = jnp.tanh(lax.dot_general(z1, w1_ref[...], (((1,), (1,)), ((), ())),
                                      precision=_HIGH) + b1_ref[...][None, :])
        t2 = jnp.tanh(lax.dot_general(z2, w1_ref[...], (((1,), (1,)), ((), ())),
                                      precision=_HIGH) + b1_ref[...][None, :])
        w2row = w2_ref[...][0]
        s1 = jnp.sum(t1 * w2row[None, :]) / N
        s2 = jnp.sum(t2 * w2row[None, :]) / N
        m = jnp.maximum(s1, s2)
        e1 = jnp.exp(s1 - m)
        e2 = jnp.exp(s2 - m)
        b1 = e1 / (e1 + e2)
        b2 = e2 / (e1 + e2)
        hfin = b1 * z1 + b2 * z2
        out_ref[...] = lax.dot_general(hfin, pw_ref[...], (((1,), (1,)), ((), ())),
                                       precision=_HIGH) + pb_ref[...][None, :]

    return pl.pallas_call(
        body,
        out_shape=jax.ShapeDtypeStruct((N, OUT), jnp.float32),
    )(acc, bias_g, sa_w1, sa_b1, sa_w2, pred_w, pred_b)


def kernel(x, edge_index1, edge_weight1, edge_index2, edge_weight2, fc_w,
           attn_l, attn_r, bias_g, sa_w1, sa_b1, sa_w2, pred_w, pred_b):
    head_of = jnp.arange(HD, dtype=jnp.int32)[:, None] // D
    mask = (head_of == jnp.arange(H, dtype=jnp.int32)[None, :]).astype(jnp.float32)
    al_mat = attn_l.reshape(HD)[:, None] * mask
    ar_mat = attn_r.reshape(HD)[:, None] * mask

    fs, eld, erd = _dense_pre(x, fc_w, al_mat, ar_mat)
    pk1 = jnp.concatenate(
        [edge_index1,
         lax.bitcast_convert_type(edge_weight1, jnp.int32)[None, :]], axis=0)
    pk2 = jnp.concatenate(
        [edge_index2,
         lax.bitcast_convert_type(edge_weight2, jnp.int32)[None, :]], axis=0)
    acc = _sc_aggregate(fs, eld, erd, pk1, pk2)
    return _dense_post(acc, bias_g, sa_w1, sa_b1, sa_w2, pred_w, pred_b)
